# baseline (speedup 1.0000x reference)
import functools

import jax
import jax.numpy as jnp
from jax import lax
from jax.experimental import pallas as pl
from jax.experimental.pallas import tpu as pltpu
from jax.experimental.pallas import tpu_sc as plsc

D_EMBED = 64
SCALE = 8.0
CHUNK = 128
LANES = 16


@functools.lru_cache(maxsize=None)
def _make(batch, hist, vocab):
    info = plsc.get_sparse_core_info()
    nc, ns = info.num_cores, info.num_subcores
    nw = nc * ns
    b_total = batch * hist
    assert b_total % (nw * CHUNK) == 0
    n_chunks = b_total // (nw * CHUNK)
    b_per_w = n_chunks * CHUNK

    mesh = plsc.VectorSubcoreMesh(core_axis_name="c", subcore_axis_name="s")

    @functools.partial(
        pl.kernel,
        out_type=jax.ShapeDtypeStruct((b_total, D_EMBED), jnp.float32),
        mesh=mesh,
        scratch_types=[
            pltpu.VMEM((n_chunks, CHUNK), jnp.int32),
            pltpu.VMEM((CHUNK, D_EMBED), jnp.float32),
            pltpu.SemaphoreType.DMA,
        ],
        compiler_params=pltpu.CompilerParams(use_tc_tiling_on_sc=False),
    )
    def gather_scale(x_hbm, table_hbm, out_hbm, idx_v, rows_v, gsem):
        wid = lax.axis_index("s") * nc + lax.axis_index("c")
        base = wid * b_per_w
        pltpu.sync_copy(x_hbm.at[wid], idx_v)

        def chunk_body(j, carry):
            pltpu.async_copy(table_hbm.at[idx_v.at[j]], rows_v, gsem).wait()

            def row_body(r, c2):
                for c in range(D_EMBED // LANES):
                    sl = pl.ds(c * LANES, LANES)
                    rows_v[r, sl] = rows_v[r, sl] * SCALE
                return c2

            lax.fori_loop(0, CHUNK, row_body, 0)
            pltpu.sync_copy(
                rows_v, out_hbm.at[pl.ds(base + j * CHUNK, CHUNK)]
            )
            return carry

        lax.fori_loop(0, n_chunks, chunk_body, 0)

    return gather_scale


def kernel(x, table):
    batch, hist = x.shape
    vocab = table.shape[0]
    info = plsc.get_sparse_core_info()
    nw = info.num_cores * info.num_subcores
    fn = _make(batch, hist, vocab)
    xw = x.reshape(-1).astype(jnp.int32).reshape(nw, -1, CHUNK)
    out = fn(xw, table)
    return out.reshape(batch, hist, D_EMBED)


# speedup vs baseline: 1.2016x; 1.2016x over previous
import functools

import jax
import jax.numpy as jnp
from jax import lax
from jax.experimental import pallas as pl
from jax.experimental.pallas import tpu as pltpu
from jax.experimental.pallas import tpu_sc as plsc

D_EMBED = 64
SCALE = 8.0
CHUNK = 128
LANES = 16
NBUF = 4
ROWS_PER_ITER = 8


@functools.lru_cache(maxsize=None)
def _make(batch, hist, vocab):
    info = plsc.get_sparse_core_info()
    nc, ns = info.num_cores, info.num_subcores
    nw = nc * ns
    b_total = batch * hist
    assert b_total % (nw * CHUNK) == 0
    n_chunks = b_total // (nw * CHUNK)
    assert n_chunks >= NBUF + 2
    b_per_w = n_chunks * CHUNK

    mesh = plsc.VectorSubcoreMesh(core_axis_name="c", subcore_axis_name="s")

    @functools.partial(
        pl.kernel,
        out_type=jax.ShapeDtypeStruct((b_total, D_EMBED), jnp.float32),
        mesh=mesh,
        scratch_types=[
            pltpu.VMEM((n_chunks, CHUNK), jnp.int32),
            [pltpu.VMEM((CHUNK, D_EMBED), jnp.float32) for _ in range(NBUF)],
            [pltpu.SemaphoreType.DMA for _ in range(NBUF)],
            [pltpu.SemaphoreType.DMA for _ in range(NBUF)],
        ],
        compiler_params=pltpu.CompilerParams(use_tc_tiling_on_sc=False),
    )
    def gather_scale(x_hbm, table_hbm, out_hbm, idx_v, rows, gsems, ssems):
        wid = lax.axis_index("s") * nc + lax.axis_index("c")
        base = wid * b_per_w
        pltpu.sync_copy(x_hbm.at[wid], idx_v)

        def start_gather(j, b):
            pltpu.async_copy(table_hbm.at[idx_v.at[j]], rows[b], gsems[b])

        def wait_gather(j, b):
            pltpu.make_async_copy(
                table_hbm.at[idx_v.at[j]], rows[b], gsems[b]
            ).wait()

        def out_slice(j):
            return out_hbm.at[pl.ds(base + j * CHUNK, CHUNK)]

        def start_store(j, b):
            pltpu.async_copy(rows[b], out_slice(j), ssems[b])

        def wait_store(j, b):
            pltpu.make_async_copy(rows[b], out_slice(j), ssems[b]).wait()

        def scale(b):
            buf = rows[b]

            def body(r8, carry):
                r0 = r8 * ROWS_PER_ITER
                for k in range(ROWS_PER_ITER):
                    for c in range(D_EMBED // LANES):
                        sl = pl.ds(c * LANES, LANES)
                        buf[r0 + k, sl] = buf[r0 + k, sl] * SCALE
                return carry

            lax.fori_loop(0, CHUNK // ROWS_PER_ITER, body, 0)

        start_gather(0, 0)
        start_gather(1, 1)
        for i in range(2):
            wait_gather(i, i)
            start_gather(i + 2, i + 2)
            scale(i)
            start_store(i, i)

        n_main = (n_chunks - NBUF) // NBUF

        def block(g, carry):
            j0 = 2 + g * NBUF
            for b0 in range(NBUF):
                i = j0 + b0
                b = (2 + b0) % NBUF
                wait_gather(i, b)
                b2 = (b + 2) % NBUF
                wait_store(i - 2, b2)
                start_gather(i + 2, b2)
                scale(b)
                start_store(i, b)
            return carry

        lax.fori_loop(0, n_main, block, 0)

        for i in range(n_chunks - 2, n_chunks):
            b = i % NBUF
            wait_gather(i, b)
            scale(b)
            start_store(i, b)

        for i in range(n_chunks - NBUF, n_chunks):
            wait_store(i, i % NBUF)

    return gather_scale


def kernel(x, table):
    batch, hist = x.shape
    vocab = table.shape[0]
    info = plsc.get_sparse_core_info()
    nw = info.num_cores * info.num_subcores
    fn = _make(batch, hist, vocab)
    xw = x.reshape(-1).astype(jnp.int32).reshape(nw, -1, CHUNK)
    out = fn(xw, table)
    return out.reshape(batch, hist, D_EMBED)
